# baseline (speedup 1.0000x reference)
import functools

import jax
import jax.numpy as jnp
from jax import lax
from jax.experimental import pallas as pl
from jax.experimental.pallas import tpu as pltpu
from jax.experimental.pallas import tpu_sc as plsc

NC = 2
NS = 16
NW = NC * NS
L = 16


def _scores_body(E, D, EPW, CHUNK, NCH,
                 h_hbm, src_hbm, dst_hbm, out_hbm,
                 sidx_v, didx_v,
                 srows0, drows0, srows1, drows1,
                 outall_v,
                 sem_s0a, sem_s0b, sem_d0a, sem_d0b,
                 sem_s1a, sem_s1b, sem_d1a, sem_d1b):
    wid = lax.axis_index("s") * NC + lax.axis_index("c")
    base = wid * EPW
    H = CHUNK // 2
    bufs = ((srows0, drows0, sem_s0a, sem_s0b, sem_d0a, sem_d0b),
            (srows1, drows1, sem_s1a, sem_s1b, sem_d1a, sem_d1b))

    def _copies(g, srows, drows, sem_sa, sem_sb, sem_da, sem_db):
        gb = g * CHUNK
        return (
            pltpu.make_async_copy(h_hbm.at[sidx_v.at[pl.ds(gb, H)]],
                                  srows.at[pl.ds(0, H)], sem_sa),
            pltpu.make_async_copy(h_hbm.at[sidx_v.at[pl.ds(gb + H, H)]],
                                  srows.at[pl.ds(H, H)], sem_sb),
            pltpu.make_async_copy(h_hbm.at[didx_v.at[pl.ds(gb, H)]],
                                  drows.at[pl.ds(0, H)], sem_da),
            pltpu.make_async_copy(h_hbm.at[didx_v.at[pl.ds(gb + H, H)]],
                                  drows.at[pl.ds(H, H)], sem_db),
        )

    def issue(g, *buf):
        for c in _copies(g, *buf):
            c.start()

    def wait(g, *buf):
        for c in _copies(g, *buf):
            c.wait()

    lane = lax.iota(jnp.int32, L)

    def one_group(g, srows, drows, j):
        jb = j * L
        r = None
        for jj in range(L):
            e = jb + jj
            a = srows[e, pl.ds(0, L)] * drows[e, pl.ds(0, L)]
            b2 = srows[e, pl.ds(L, L)] * drows[e, pl.ds(L, L)]
            for cc in range(2, D // L, 2):
                a = a + (srows[e, pl.ds(cc * L, L)]
                         * drows[e, pl.ds(cc * L, L)])
                b2 = b2 + (srows[e, pl.ds((cc + 1) * L, L)]
                           * drows[e, pl.ds((cc + 1) * L, L)])
            a = a + b2
            s = jnp.broadcast_to(jnp.sum(a), (L,))
            r = s if r is None else jnp.where(lane == jj, s, r)
        outall_v[pl.ds(g * CHUNK + jb, L)] = 1.0 / (1.0 + jnp.exp(-r))

    def compute(g, srows, drows):
        for j in range(CHUNK // L):
            one_group(g, srows, drows, j)

    pltpu.sync_copy(src_hbm.at[pl.ds(base, EPW)], sidx_v)
    pltpu.sync_copy(dst_hbm.at[pl.ds(base, EPW)], didx_v)

    issue(0, *bufs[0])

    def pair_body(g2, carry):
        g = g2 * 2
        for b in range(2):
            gg = g + b
            wait(gg, *bufs[b])
            issue(gg + 1, *bufs[1 - b])
            compute(gg, bufs[b][0], bufs[b][1])
        return carry

    lax.fori_loop(0, (NCH - 1) // 2, pair_body, 0, unroll=False)
    last = NCH - 1
    wait(last, *bufs[last % 2])
    compute(last, bufs[last % 2][0], bufs[last % 2][1])

    pltpu.sync_copy(outall_v, out_hbm.at[pl.ds(base, EPW)])


def kernel(h, edge_index):
    N, D = h.shape
    E = edge_index.shape[1]
    EPW = E // NW
    CHUNK = 80
    NCH = EPW // CHUNK
    assert EPW * NW == E and NCH * CHUNK == EPW and D % L == 0
    assert NCH % 2 == 1

    src = edge_index[0]
    dst = edge_index[1]

    mesh = plsc.VectorSubcoreMesh(core_axis_name="c", subcore_axis_name="s",
                                  num_cores=NC, num_subcores=NS)
    body = functools.partial(_scores_body, E, D, EPW, CHUNK, NCH)
    f = pl.kernel(
        body,
        out_type=jax.ShapeDtypeStruct((E,), jnp.float32),
        mesh=mesh,
        compiler_params=pltpu.CompilerParams(needs_layout_passes=False),
        scratch_types=[
            pltpu.VMEM((EPW,), jnp.int32),
            pltpu.VMEM((EPW,), jnp.int32),
            pltpu.VMEM((CHUNK, D), jnp.float32),
            pltpu.VMEM((CHUNK, D), jnp.float32),
            pltpu.VMEM((CHUNK, D), jnp.float32),
            pltpu.VMEM((CHUNK, D), jnp.float32),
            pltpu.VMEM((EPW,), jnp.float32),
            pltpu.SemaphoreType.DMA,
            pltpu.SemaphoreType.DMA,
            pltpu.SemaphoreType.DMA,
            pltpu.SemaphoreType.DMA,
            pltpu.SemaphoreType.DMA,
            pltpu.SemaphoreType.DMA,
            pltpu.SemaphoreType.DMA,
            pltpu.SemaphoreType.DMA,
        ],
    )
    return f(h, src, dst)


# speedup vs baseline: 1.0417x; 1.0417x over previous
import functools

import jax
import jax.numpy as jnp
from jax import lax
from jax.experimental import pallas as pl
from jax.experimental.pallas import tpu as pltpu
from jax.experimental.pallas import tpu_sc as plsc

NC = 2
NS = 16
NW = NC * NS
L = 16


def _scores_body(E, D, EPW, CHUNK, NCH,
                 h_hbm, src_hbm, dst_hbm, out_hbm,
                 sidx_v, didx_v,
                 srows0, drows0, srows1, drows1,
                 outall_v,
                 sem_s0a, sem_s0b, sem_d0a, sem_d0b,
                 sem_s1a, sem_s1b, sem_d1a, sem_d1b):
    wid = lax.axis_index("s") * NC + lax.axis_index("c")
    base = wid * EPW
    H = CHUNK // 2
    bufs = ((srows0, drows0, sem_s0a, sem_s0b, sem_d0a, sem_d0b),
            (srows1, drows1, sem_s1a, sem_s1b, sem_d1a, sem_d1b))

    def _copies(g, srows, drows, sem_sa, sem_sb, sem_da, sem_db):
        gb = g * CHUNK
        return (
            pltpu.make_async_copy(h_hbm.at[sidx_v.at[pl.ds(gb, H)]],
                                  srows.at[pl.ds(0, H)], sem_sa),
            pltpu.make_async_copy(h_hbm.at[sidx_v.at[pl.ds(gb + H, H)]],
                                  srows.at[pl.ds(H, H)], sem_sb),
            pltpu.make_async_copy(h_hbm.at[didx_v.at[pl.ds(gb, H)]],
                                  drows.at[pl.ds(0, H)], sem_da),
            pltpu.make_async_copy(h_hbm.at[didx_v.at[pl.ds(gb + H, H)]],
                                  drows.at[pl.ds(H, H)], sem_db),
        )

    def issue(g, *buf):
        for c in _copies(g, *buf):
            c.start()

    def wait(g, *buf):
        for c in _copies(g, *buf):
            c.wait()

    lane = lax.iota(jnp.int32, L)

    def one_group(g, srows, drows, j):
        jb = j * L
        r = None
        for jj in range(L):
            e = jb + jj
            a = srows[e, pl.ds(0, L)] * drows[e, pl.ds(0, L)]
            b2 = srows[e, pl.ds(L, L)] * drows[e, pl.ds(L, L)]
            for cc in range(2, D // L, 2):
                a = a + (srows[e, pl.ds(cc * L, L)]
                         * drows[e, pl.ds(cc * L, L)])
                b2 = b2 + (srows[e, pl.ds((cc + 1) * L, L)]
                           * drows[e, pl.ds((cc + 1) * L, L)])
            a = a + b2
            s = jnp.broadcast_to(jnp.sum(a), (L,))
            r = s if r is None else jnp.where(lane == jj, s, r)
        outall_v[pl.ds(g * CHUNK + jb, L)] = r

    def compute(g, srows, drows):
        for j in range(CHUNK // L):
            one_group(g, srows, drows, j)

    pltpu.sync_copy(src_hbm.at[pl.ds(base, EPW)], sidx_v)
    pltpu.sync_copy(dst_hbm.at[pl.ds(base, EPW)], didx_v)

    issue(0, *bufs[0])

    def pair_body(g2, carry):
        g = g2 * 2
        for b in range(2):
            gg = g + b
            wait(gg, *bufs[b])
            issue(gg + 1, *bufs[1 - b])
            compute(gg, bufs[b][0], bufs[b][1])
        return carry

    lax.fori_loop(0, (NCH - 1) // 2, pair_body, 0, unroll=False)
    last = NCH - 1
    wait(last, *bufs[last % 2])
    compute(last, bufs[last % 2][0], bufs[last % 2][1])

    def sig_body(i, c):
        v = outall_v[pl.ds(i * L, L)]
        outall_v[pl.ds(i * L, L)] = 1.0 / (1.0 + jnp.exp(-v))
        return c

    lax.fori_loop(0, EPW // L, sig_body, 0, unroll=False)

    pltpu.sync_copy(outall_v, out_hbm.at[pl.ds(base, EPW)])


def kernel(h, edge_index):
    N, D = h.shape
    E = edge_index.shape[1]
    EPW = E // NW
    CHUNK = 80
    NCH = EPW // CHUNK
    assert EPW * NW == E and NCH * CHUNK == EPW and D % L == 0
    assert NCH % 2 == 1

    src = edge_index[0]
    dst = edge_index[1]

    mesh = plsc.VectorSubcoreMesh(core_axis_name="c", subcore_axis_name="s",
                                  num_cores=NC, num_subcores=NS)
    body = functools.partial(_scores_body, E, D, EPW, CHUNK, NCH)
    f = pl.kernel(
        body,
        out_type=jax.ShapeDtypeStruct((E,), jnp.float32),
        mesh=mesh,
        compiler_params=pltpu.CompilerParams(needs_layout_passes=False),
        scratch_types=[
            pltpu.VMEM((EPW,), jnp.int32),
            pltpu.VMEM((EPW,), jnp.int32),
            pltpu.VMEM((CHUNK, D), jnp.float32),
            pltpu.VMEM((CHUNK, D), jnp.float32),
            pltpu.VMEM((CHUNK, D), jnp.float32),
            pltpu.VMEM((CHUNK, D), jnp.float32),
            pltpu.VMEM((EPW,), jnp.float32),
            pltpu.SemaphoreType.DMA,
            pltpu.SemaphoreType.DMA,
            pltpu.SemaphoreType.DMA,
            pltpu.SemaphoreType.DMA,
            pltpu.SemaphoreType.DMA,
            pltpu.SemaphoreType.DMA,
            pltpu.SemaphoreType.DMA,
            pltpu.SemaphoreType.DMA,
        ],
    )
    return f(h, src, dst)


# speedup vs baseline: 1.0591x; 1.0167x over previous
import functools

import jax
import jax.numpy as jnp
from jax import lax
from jax.experimental import pallas as pl
from jax.experimental.pallas import tpu as pltpu
from jax.experimental.pallas import tpu_sc as plsc

NC = 2
NS = 16
NW = NC * NS
L = 16


def _scores_body(E, D, EPW, CHUNK, NCH,
                 h_hbm, src_hbm, dst_hbm, out_hbm,
                 sidx_v, didx_v,
                 srows0, drows0, srows1, drows1,
                 outall_v,
                 sem_s0a, sem_s0b, sem_d0a, sem_d0b,
                 sem_s1a, sem_s1b, sem_d1a, sem_d1b):
    wid = lax.axis_index("s") * NC + lax.axis_index("c")
    base = wid * EPW
    H = CHUNK // 2
    bufs = ((srows0, drows0, sem_s0a, sem_s0b, sem_d0a, sem_d0b),
            (srows1, drows1, sem_s1a, sem_s1b, sem_d1a, sem_d1b))

    def _copies(g, srows, drows, sem_sa, sem_sb, sem_da, sem_db):
        gb = g * CHUNK
        return (
            pltpu.make_async_copy(h_hbm.at[sidx_v.at[pl.ds(gb, H)]],
                                  srows.at[pl.ds(0, H)], sem_sa),
            pltpu.make_async_copy(h_hbm.at[sidx_v.at[pl.ds(gb + H, H)]],
                                  srows.at[pl.ds(H, H)], sem_sb),
            pltpu.make_async_copy(h_hbm.at[didx_v.at[pl.ds(gb, H)]],
                                  drows.at[pl.ds(0, H)], sem_da),
            pltpu.make_async_copy(h_hbm.at[didx_v.at[pl.ds(gb + H, H)]],
                                  drows.at[pl.ds(H, H)], sem_db),
        )

    def issue(g, *buf):
        for c in _copies(g, *buf):
            c.start()

    def wait(g, *buf):
        for c in _copies(g, *buf):
            c.wait()

    lane = lax.iota(jnp.int32, L)

    def one_group(g, srows, drows, j):
        jb = j * L
        r = None
        for jj in range(L):
            e = jb + jj
            a = srows[e, pl.ds(0, L)] * drows[e, pl.ds(0, L)]
            b2 = srows[e, pl.ds(L, L)] * drows[e, pl.ds(L, L)]
            for cc in range(2, D // L, 2):
                a = a + (srows[e, pl.ds(cc * L, L)]
                         * drows[e, pl.ds(cc * L, L)])
                b2 = b2 + (srows[e, pl.ds((cc + 1) * L, L)]
                           * drows[e, pl.ds((cc + 1) * L, L)])
            a = a + b2
            s = jnp.broadcast_to(jnp.sum(a), (L,))
            r = s if r is None else jnp.where(lane == jj, s, r)
        outall_v[pl.ds(g * CHUNK + jb, L)] = r

    def compute(g, srows, drows):
        for j in range(CHUNK // L):
            one_group(g, srows, drows, j)

    pltpu.sync_copy(src_hbm.at[pl.ds(base, EPW)], sidx_v)
    pltpu.sync_copy(dst_hbm.at[pl.ds(base, EPW)], didx_v)

    issue(0, *bufs[0])

    def pair_body(g2, carry):
        g = g2 * 2
        for b in range(2):
            gg = g + b
            wait(gg, *bufs[b])
            issue(gg + 1, *bufs[1 - b])
            compute(gg, bufs[b][0], bufs[b][1])
        return carry

    lax.fori_loop(0, (NCH - 1) // 2, pair_body, 0, unroll=False)
    last = NCH - 1
    wait(last, *bufs[last % 2])
    compute(last, bufs[last % 2][0], bufs[last % 2][1])

    def sig_body(i, c):
        v = outall_v[pl.ds(i * L, L)]
        outall_v[pl.ds(i * L, L)] = 1.0 / (1.0 + jnp.exp(-v))
        return c

    lax.fori_loop(0, EPW // L, sig_body, 0, unroll=8)

    pltpu.sync_copy(outall_v, out_hbm.at[pl.ds(base, EPW)])


def kernel(h, edge_index):
    N, D = h.shape
    E = edge_index.shape[1]
    EPW = E // NW
    CHUNK = 80
    NCH = EPW // CHUNK
    assert EPW * NW == E and NCH * CHUNK == EPW and D % L == 0
    assert NCH % 2 == 1

    src = edge_index[0]
    dst = edge_index[1]

    mesh = plsc.VectorSubcoreMesh(core_axis_name="c", subcore_axis_name="s",
                                  num_cores=NC, num_subcores=NS)
    body = functools.partial(_scores_body, E, D, EPW, CHUNK, NCH)
    f = pl.kernel(
        body,
        out_type=jax.ShapeDtypeStruct((E,), jnp.float32),
        mesh=mesh,
        compiler_params=pltpu.CompilerParams(needs_layout_passes=False),
        scratch_types=[
            pltpu.VMEM((EPW,), jnp.int32),
            pltpu.VMEM((EPW,), jnp.int32),
            pltpu.VMEM((CHUNK, D), jnp.float32),
            pltpu.VMEM((CHUNK, D), jnp.float32),
            pltpu.VMEM((CHUNK, D), jnp.float32),
            pltpu.VMEM((CHUNK, D), jnp.float32),
            pltpu.VMEM((EPW,), jnp.float32),
            pltpu.SemaphoreType.DMA,
            pltpu.SemaphoreType.DMA,
            pltpu.SemaphoreType.DMA,
            pltpu.SemaphoreType.DMA,
            pltpu.SemaphoreType.DMA,
            pltpu.SemaphoreType.DMA,
            pltpu.SemaphoreType.DMA,
            pltpu.SemaphoreType.DMA,
        ],
    )
    return f(h, src, dst)
